# Initial kernel scaffold; baseline (speedup 1.0000x reference)
import jax
import jax.numpy as jnp
from jax.experimental import pallas as pl


def kernel(indices, emb_dim, table):
    raise NotImplementedError("write your pallas kernel here")


# speedup vs baseline: 3.4320x; 3.4320x over previous
import functools

import jax
import jax.numpy as jnp
from jax import lax
from jax.experimental import pallas as pl
from jax.experimental.pallas import tpu as pltpu
from jax.experimental.pallas import tpu_sc as plsc

NC = 2
NS = 16
NW = NC * NS
CH = 128
K = 4
GROUP = CH * K
NBUF = 2


def _make_gather(V, D, n_chunks):
    n_groups = n_chunks // K
    bpw = n_chunks * CH
    mesh = plsc.VectorSubcoreMesh(
        core_axis_name="c", subcore_axis_name="s", num_cores=NC, num_subcores=NS
    )

    @functools.partial(
        pl.kernel,
        out_type=jax.ShapeDtypeStruct((NW * bpw, D), jnp.float32),
        mesh=mesh,
        compiler_params=pltpu.CompilerParams(use_tc_tiling_on_sc=False),
        scratch_types=[
            pltpu.VMEM((n_chunks, CH), jnp.int32),
            pltpu.VMEM((NBUF, GROUP, D), jnp.float32),
            pltpu.SemaphoreType.DMA,
            pltpu.SemaphoreType.DMA,
            pltpu.SemaphoreType.DMA,
            pltpu.SemaphoreType.DMA,
        ],
    )
    def gather_kernel(idx_hbm, table_hbm, out_hbm, idx_v, rows_v, g0, g1, w0, w1):
        gsem = [g0, g1]
        wsem = [w0, w1]
        wid = lax.axis_index("s") * NC + lax.axis_index("c")
        base = wid * bpw
        pltpu.sync_copy(idx_hbm.at[wid], idx_v)

        def fire(g, b):
            for j in range(K):
                pltpu.async_copy(
                    table_hbm.at[idx_v.at[g * K + j]],
                    rows_v.at[b, pl.ds(j * CH, CH)],
                    gsem[b],
                )

        def drain_gathers(b):
            pltpu.make_async_copy(
                out_hbm.at[pl.ds(0, GROUP)], rows_v.at[b], gsem[b]
            ).wait()

        def start_write(g, b):
            pltpu.async_copy(
                rows_v.at[b], out_hbm.at[pl.ds(base + g * GROUP, GROUP)], wsem[b]
            )

        def wait_write(b):
            pltpu.make_async_copy(
                rows_v.at[b], out_hbm.at[pl.ds(0, GROUP)], wsem[b]
            ).wait()

        def do_group(g, b):
            drain_gathers(b)
            start_write(g, b)
            gn = g + 1
            bn = b ^ 1

            @pl.when(gn < n_groups)
            def _():
                @pl.when(gn >= NBUF)
                def _():
                    wait_write(bn)

                fire(gn, bn)

        fire(0, 0)

        def body(t, carry):
            do_group(t * NBUF, 0)
            do_group(t * NBUF + 1, 1)
            return carry

        lax.fori_loop(0, n_groups // NBUF, body, 0)
        wait_write(0)
        wait_write(1)

    return gather_kernel


def kernel(indices, emb_dim, table):
    n1, n2 = indices.shape
    V, D = table.shape
    B = n1 * n2
    assert B % (NW * GROUP) == 0
    n_chunks = B // (NW * CH)

    gate = jnp.where(
        jnp.asarray(emb_dim) == D, jnp.float32(1.0), jnp.float32(jnp.nan)
    ).astype(table.dtype)
    table_gated = (table * gate).astype(jnp.float32)

    idx_blocks = indices.reshape(NW, n_chunks, CH)
    out = _make_gather(V, D, n_chunks)(idx_blocks, table_gated)
    return out.reshape(n1, n2, D)
